# baseline (speedup 1.0000x reference)
import functools

import jax
import jax.numpy as jnp
from jax import lax
from jax.experimental import pallas as pl
from jax.experimental.pallas import tpu as pltpu
from jax.experimental.pallas import tpu_sc as plsc

_NUM_FIELDS = 26
_BATCH = 4096
_DIM = 32
_NBUF = 4


def _build():
    info = plsc.get_sparse_core_info()
    nc, ns = info.num_cores, info.num_subcores
    nw = nc * ns
    bpw = _BATCH // nw
    lanes = 16

    mesh = plsc.VectorSubcoreMesh(core_axis_name="c", subcore_axis_name="s")
    out_type = tuple(
        jax.ShapeDtypeStruct((_BATCH // 32, 8 * 128), jnp.float32)
        for _ in range(_NUM_FIELDS)
    )
    scratch = (
        [pltpu.VMEM((bpw,), jnp.int32) for _ in range(_NUM_FIELDS)]
        + [pltpu.VMEM((bpw, _DIM), jnp.float32) for _ in range(_NBUF)]
        + [pltpu.VMEM((4, 8 * 128), jnp.float32) for _ in range(2)]
        + [pltpu.SemaphoreType.DMA for _ in range(3)]
    )

    @functools.partial(pl.kernel, mesh=mesh, out_type=out_type,
                       scratch_types=scratch,
                       compiler_params=pltpu.CompilerParams(
                           use_tc_tiling_on_sc=False,
                           needs_layout_passes=False))
    def body(*refs):
        idx_refs = refs[:_NUM_FIELDS]
        tab3 = refs[_NUM_FIELDS]
        out_refs = refs[_NUM_FIELDS + 1:2 * _NUM_FIELDS + 1]
        sc = refs[2 * _NUM_FIELDS + 1:]
        idx_v = sc[:_NUM_FIELDS]
        rows_v = sc[_NUM_FIELDS:_NUM_FIELDS + _NBUF]
        tbuf = sc[_NUM_FIELDS + _NBUF:_NUM_FIELDS + _NBUF + 2]
        isem, gsem, wsem = sc[_NUM_FIELDS + _NBUF + 2:]

        wid = lax.axis_index("s") * nc + lax.axis_index("c")
        base = wid * bpw
        sl = pl.ds(base, bpw)

        ih = [pltpu.async_copy(idx_refs[f].at[sl], idx_v[f], isem)
              for f in range(_NUM_FIELDS)]
        gh = [None] * _NUM_FIELDS

        def fire(f):
            ih[f].wait()
            gh[f] = pltpu.async_copy(tab3.at[f].at[idx_v[f]],
                                     rows_v[f % _NBUF], gsem)

        def scramble(f):
            rv = rows_v[f % _NBUF]
            tb = tbuf[f % 2]
            def cloop(c, _):
                cb = c // 8
                s = c % 8
                def jloop(jv, _):
                    ridx = lax.iota(jnp.int32, lanes) + jv * lanes
                    cidx = jnp.full((lanes,), c, jnp.int32)
                    vals = plsc.load_gather(rv, [ridx, cidx])
                    tb[cb, pl.ds(s * 128 + jv * lanes, lanes)] = vals
                    return 0
                lax.fori_loop(0, bpw // lanes, jloop, 0)
                return 0
            lax.fori_loop(0, _DIM, cloop, 0)

        def out_start(f):
            tb = tbuf[f % 2]
            for cb in range(4):
                pltpu.async_copy(tb.at[cb], out_refs[f].at[cb * 32 + wid],
                                 wsem)

        def out_wait(f):
            tb = tbuf[f % 2]
            for cb in range(4):
                pltpu.make_async_copy(
                    tb.at[cb], out_refs[f].at[cb * 32 + wid], wsem).wait()

        for f in range(min(_NBUF, _NUM_FIELDS)):
            fire(f)
        for f in range(_NUM_FIELDS):
            gh[f].wait()
            if f >= 2:
                out_wait(f - 2)
            scramble(f)
            out_start(f)
            nxt = f + _NBUF
            if nxt < _NUM_FIELDS:
                fire(nxt)
        out_wait(_NUM_FIELDS - 2)
        out_wait(_NUM_FIELDS - 1)

    return body


_sc_kernel = _build()


def kernel(idx_00, idx_01, idx_02, idx_03, idx_04, idx_05, idx_06, idx_07, idx_08, idx_09, idx_10, idx_11, idx_12, idx_13, idx_14, idx_15, idx_16, idx_17, idx_18, idx_19, idx_20, idx_21, idx_22, idx_23, idx_24, idx_25, table_00, table_01, table_02, table_03, table_04, table_05, table_06, table_07, table_08, table_09, table_10, table_11, table_12, table_13, table_14, table_15, table_16, table_17, table_18, table_19, table_20, table_21, table_22, table_23, table_24, table_25):
    tab3 = jnp.stack((
        table_00, table_01, table_02, table_03, table_04, table_05,
        table_06, table_07, table_08, table_09, table_10, table_11,
        table_12, table_13, table_14, table_15, table_16, table_17,
        table_18, table_19, table_20, table_21, table_22, table_23,
        table_24, table_25,
    ))
    outs = _sc_kernel(
        idx_00, idx_01, idx_02, idx_03, idx_04, idx_05, idx_06, idx_07,
        idx_08, idx_09, idx_10, idx_11, idx_12, idx_13, idx_14, idx_15,
        idx_16, idx_17, idx_18, idx_19, idx_20, idx_21, idx_22, idx_23,
        idx_24, idx_25,
        tab3,
    )
    return tuple(
        o.reshape(4, 32, 8, 128).transpose(1, 3, 0, 2).reshape(_BATCH, _DIM)
        for o in outs
    )


# speedup vs baseline: 4.9005x; 4.9005x over previous
import functools

import jax
import jax.numpy as jnp
from jax import lax
from jax.experimental import pallas as pl
from jax.experimental.pallas import tpu as pltpu
from jax.experimental.pallas import tpu_sc as plsc

_NUM_FIELDS = 26
_BATCH = 4096
_DIM = 32
_NBUF = 4
_GROUP = 2


def _build(nf):
    info = plsc.get_sparse_core_info()
    nc, ns = info.num_cores, info.num_subcores
    nw = nc * ns
    bpw = _BATCH // nw
    lanes = 16

    mesh = plsc.VectorSubcoreMesh(core_axis_name="c", subcore_axis_name="s")
    out_type = tuple(
        jax.ShapeDtypeStruct((_BATCH // 32, 8 * 128), jnp.float32)
        for _ in range(nf)
    )
    scratch = (
        [pltpu.VMEM((bpw,), jnp.int32) for _ in range(nf)]
        + [pltpu.VMEM((bpw, _DIM), jnp.float32) for _ in range(_NBUF)]
        + [pltpu.VMEM((4, 8 * 128), jnp.float32) for _ in range(2)]
        + [pltpu.SemaphoreType.DMA for _ in range(3)]
    )

    @functools.partial(pl.kernel, mesh=mesh, out_type=out_type,
                       scratch_types=scratch,
                       compiler_params=pltpu.CompilerParams(
                           use_tc_tiling_on_sc=False,
                           needs_layout_passes=False))
    def body(*refs):
        idx_refs = refs[:nf]
        tab_refs = refs[nf:2 * nf]
        out_refs = refs[2 * nf:3 * nf]
        sc = refs[3 * nf:]
        idx_v = sc[:nf]
        rows_v = sc[nf:nf + _NBUF]
        tbuf = sc[nf + _NBUF:nf + _NBUF + 2]
        isem, gsem, wsem = sc[nf + _NBUF + 2:]

        wid = lax.axis_index("s") * nc + lax.axis_index("c")
        base = wid * bpw
        sl = pl.ds(base, bpw)

        ih = [pltpu.async_copy(idx_refs[f].at[sl], idx_v[f], isem)
              for f in range(nf)]
        gh = [None] * nf

        def fire(f):
            ih[f].wait()
            gh[f] = pltpu.async_copy(tab_refs[f].at[idx_v[f]],
                                     rows_v[f % _NBUF], gsem)

        def scramble(f):
            rv = rows_v[f % _NBUF]
            tb = tbuf[f % 2]
            def cloop(c, _):
                cb = c // 8
                s = c % 8
                def jloop(jv, _):
                    ridx = lax.iota(jnp.int32, lanes) + jv * lanes
                    cidx = jnp.full((lanes,), c, jnp.int32)
                    vals = plsc.load_gather(rv, [ridx, cidx])
                    tb[cb, pl.ds(s * 128 + jv * lanes, lanes)] = vals
                    return 0
                lax.fori_loop(0, bpw // lanes, jloop, 0)
                return 0
            lax.fori_loop(0, _DIM, cloop, 0)

        def out_start(f):
            tb = tbuf[f % 2]
            for cb in range(4):
                pltpu.async_copy(tb.at[cb], out_refs[f].at[cb * 32 + wid],
                                 wsem)

        def out_wait(f):
            tb = tbuf[f % 2]
            for cb in range(4):
                pltpu.make_async_copy(
                    tb.at[cb], out_refs[f].at[cb * 32 + wid], wsem).wait()

        for f in range(min(_NBUF, nf)):
            fire(f)
        for f in range(nf):
            gh[f].wait()
            if f >= 2:
                out_wait(f - 2)
            scramble(f)
            out_start(f)
            nxt = f + _NBUF
            if nxt < nf:
                fire(nxt)
        out_wait(nf - 2)
        out_wait(nf - 1)

    return body


_sc_kernel_g = _build(_GROUP)
_sc_kernel_r = _build(_NUM_FIELDS % _GROUP) if _NUM_FIELDS % _GROUP else None


def kernel(idx_00, idx_01, idx_02, idx_03, idx_04, idx_05, idx_06, idx_07, idx_08, idx_09, idx_10, idx_11, idx_12, idx_13, idx_14, idx_15, idx_16, idx_17, idx_18, idx_19, idx_20, idx_21, idx_22, idx_23, idx_24, idx_25, table_00, table_01, table_02, table_03, table_04, table_05, table_06, table_07, table_08, table_09, table_10, table_11, table_12, table_13, table_14, table_15, table_16, table_17, table_18, table_19, table_20, table_21, table_22, table_23, table_24, table_25):
    idxs = (idx_00, idx_01, idx_02, idx_03, idx_04, idx_05, idx_06, idx_07,
            idx_08, idx_09, idx_10, idx_11, idx_12, idx_13, idx_14, idx_15,
            idx_16, idx_17, idx_18, idx_19, idx_20, idx_21, idx_22, idx_23,
            idx_24, idx_25)
    tabs = (table_00, table_01, table_02, table_03, table_04, table_05,
            table_06, table_07, table_08, table_09, table_10, table_11,
            table_12, table_13, table_14, table_15, table_16, table_17,
            table_18, table_19, table_20, table_21, table_22, table_23,
            table_24, table_25)
    outs = []
    f = 0
    while f < _NUM_FIELDS:
        n = min(_GROUP, _NUM_FIELDS - f)
        fn = _sc_kernel_g if n == _GROUP else _sc_kernel_r
        res = fn(*idxs[f:f + n], *tabs[f:f + n])
        outs.extend(res if isinstance(res, (tuple, list)) else [res])
        f += n
    return tuple(
        o.reshape(4, 32, 8, 128).transpose(1, 3, 0, 2).reshape(_BATCH, _DIM)
        for o in outs
    )
